# Initial kernel scaffold; baseline (speedup 1.0000x reference)
import jax
import jax.numpy as jnp
from jax.experimental import pallas as pl


def kernel(x, edge_index, params):
    raise NotImplementedError("write your pallas kernel here")


# speedup vs baseline: 4.2406x; 4.2406x over previous
import functools

import jax
import jax.numpy as jnp
from jax import lax
from jax.experimental import pallas as pl
from jax.experimental.pallas import tpu as pltpu
from jax.experimental.pallas import tpu_sc as plsc

N, E, HID, OUT_C = 10000, 320000, 128, 40
GROUPS, CODES, BETA, EPS = 3, 16, 0.98, 1e-5

NC, NS = 2, 16
NW = NC * NS
CH = 128
NCHUNK = -(-E // (NW * CH))
EPW = NCHUNK * CH
EPAD = EPW * NW
ACC_ROWS = 10240
RPT = ACC_ROWS // NS
DUMMY_ROW = N

_f32 = jnp.float32


def _sum_body(x_hbm, src_hbm, dst_hbm, sums_out,
              src_v, dst_v, rows_v, acc_sh, sem):
    cid = lax.axis_index("c")
    sid = lax.axis_index("s")
    wid = cid * NS + sid

    def init_row(i, _):
        for k in range(HID // 16):
            rows_v[i, pl.ds(16 * k, 16)] = jnp.zeros((16,), _f32)
        return 0
    lax.fori_loop(0, CH, init_row, 0)

    base = sid * RPT

    def zero_acc(k, _):
        pltpu.sync_copy(rows_v, acc_sh.at[pl.ds(base + k * CH, CH)])
        return 0
    lax.fori_loop(0, RPT // CH, zero_acc, 0)

    plsc.subcore_barrier()

    pltpu.sync_copy(src_hbm.at[wid], src_v)
    pltpu.sync_copy(dst_hbm.at[wid], dst_v)

    def chunk(j, _):
        pltpu.async_copy(x_hbm.at[src_v.at[j]], rows_v, sem).wait()
        pltpu.sync_copy(rows_v, acc_sh.at[dst_v.at[j]], add=True)
        return 0
    lax.fori_loop(0, NCHUNK, chunk, 0)

    plsc.subcore_barrier()

    pltpu.sync_copy(acc_sh.at[pl.ds(base, RPT)],
                    sums_out.at[cid, pl.ds(base, RPT)])


def _cnt_body(dst_hbm, cnt_out, dst_v, ones_v, cnt_sh, sem):
    cid = lax.axis_index("c")
    sid = lax.axis_index("s")
    wid = cid * NS + sid

    def fill(val):
        def body(i, _):
            for k in range(HID // 16):
                ones_v[i, pl.ds(16 * k, 16)] = jnp.full((16,), val, _f32)
            return 0
        lax.fori_loop(0, CH, body, 0)

    fill(0.0)
    base = sid * RPT

    def zero_acc(k, _):
        pltpu.sync_copy(ones_v, cnt_sh.at[pl.ds(base + k * CH, CH)])
        return 0
    lax.fori_loop(0, RPT // CH, zero_acc, 0)
    fill(1.0)

    plsc.subcore_barrier()

    pltpu.sync_copy(dst_hbm.at[wid], dst_v)

    def chunk(j, _):
        pltpu.sync_copy(ones_v, cnt_sh.at[dst_v.at[j]], add=True)
        return 0
    lax.fori_loop(0, NCHUNK, chunk, 0)

    plsc.subcore_barrier()

    pltpu.sync_copy(cnt_sh.at[pl.ds(base, RPT)],
                    cnt_out.at[cid, pl.ds(base, RPT)])


@functools.cache
def _mesh():
    return plsc.VectorSubcoreMesh(core_axis_name="c", subcore_axis_name="s",
                                  num_cores=NC, num_subcores=NS)


def _sc_segment_sum(x, src_r, dst_r):
    fn = pl.kernel(
        _sum_body,
        out_type=jax.ShapeDtypeStruct((NC, ACC_ROWS, HID), _f32),
        mesh=_mesh(),
        scratch_types=[
            pltpu.VMEM((NCHUNK, CH), jnp.int32),
            pltpu.VMEM((NCHUNK, CH), jnp.int32),
            pltpu.VMEM((CH, HID), _f32),
            pltpu.VMEM_SHARED((ACC_ROWS, HID), _f32),
            pltpu.SemaphoreType.DMA,
        ])
    return fn(x, src_r, dst_r)


def _sc_degree(dst_r):
    fn = pl.kernel(
        _cnt_body,
        out_type=jax.ShapeDtypeStruct((NC, ACC_ROWS, HID), _f32),
        mesh=_mesh(),
        scratch_types=[
            pltpu.VMEM((NCHUNK, CH), jnp.int32),
            pltpu.VMEM((CH, HID), _f32),
            pltpu.VMEM_SHARED((ACC_ROWS, HID), _f32),
            pltpu.SemaphoreType.DMA,
        ])
    return fn(dst_r)


BLK = 2000
GRID = N // BLK


def _vq(r, cb, iota):
    ids = []
    for g in range(GROUPS):
        c = cb[g * CODES:(g + 1) * CODES, :]
        d = (jnp.sum(r * r, axis=1, keepdims=True)
             - 2.0 * jnp.dot(r, c.T, preferred_element_type=_f32)
             + jnp.sum(c * c, axis=1)[None, :])
        dmin = jnp.min(d, axis=1, keepdims=True)
        idx = jnp.min(jnp.where(d <= dmin, iota, CODES), axis=1)
        q = jnp.zeros_like(r)
        for k in range(CODES):
            q = q + jnp.where((idx == k)[:, None], c[k:k + 1, :], 0.0)
        r = r - q
        ids.append(idx)
    return ids, r


def _mean_from_parts(p0, p1, c0, c1):
    cnt = jnp.maximum(c0[:, 0:1] + c1[:, 0:1], 1.0)
    return (p0[...] + p1[...]) / cnt


def _pre_body(p0, p1, c0, c1, x, wl, wr, b, pre, stats):
    i = pl.program_id(0)
    mean = _mean_from_parts(p0, p1, c0, c1)
    v = (jnp.dot(mean, wl[...], preferred_element_type=_f32)
         + jnp.dot(x[...], wr[...], preferred_element_type=_f32) + b[...])
    pre[...] = v

    @pl.when(i == 0)
    def _():
        stats[...] = jnp.zeros_like(stats)
    stats[0:1, :] += jnp.sum(v, axis=0)[None, :]
    stats[1:2, :] += jnp.sum(v * v, axis=0)[None, :]


def _post_body(pre, stats, g, bb, cb, h_out, ids_out, lacc):
    i = pl.program_id(0)
    mu = stats[0:1, :] / N
    var = stats[1:2, :] / N - mu * mu
    h = g[...] * (pre[...] - mu) / jnp.sqrt(var + EPS) + bb[...]
    h = jnp.maximum(h, 0.0)
    h_out[...] = h
    iota = lax.broadcasted_iota(jnp.int32, (BLK, CODES), 1)
    ids, r = _vq(h, cb[...], iota)
    ids_out[...] = jnp.concatenate(
        [v[:, None] for v in ids] + [jnp.zeros((BLK, 8 - GROUPS), jnp.int32)],
        axis=1)

    @pl.when(i == 0)
    def _():
        lacc[...] = jnp.zeros_like(lacc)
    lacc[0:1, 0:1] += jnp.sum(r * r)[None, None]


def _final_body(p0, p1, c0, c1, h2, wl, wr, b, cb, w, bl,
                logits, ids_out, lacc):
    i = pl.program_id(0)
    mean = _mean_from_parts(p0, p1, c0, c1)
    h = (jnp.dot(mean, wl[...], preferred_element_type=_f32)
         + jnp.dot(h2[...], wr[...], preferred_element_type=_f32) + b[...])
    iota = lax.broadcasted_iota(jnp.int32, (BLK, CODES), 1)
    ids, r = _vq(h, cb[...], iota)
    ids_out[...] = jnp.concatenate(
        [v[:, None] for v in ids] + [jnp.zeros((BLK, 8 - GROUPS), jnp.int32)],
        axis=1)
    logits[...] = jnp.dot(h, w[...], preferred_element_type=_f32) + bl[...]

    @pl.when(i == 0)
    def _():
        lacc[...] = jnp.zeros_like(lacc)
    lacc[0:1, 0:1] += jnp.sum(r * r)[None, None]


def _row_spec(width):
    return pl.BlockSpec((BLK, width), lambda i: (i, 0))


def _full_spec(rows, cols):
    return pl.BlockSpec((rows, cols), lambda i: (0, 0))


def _layer_pre(p0, p1, c0, c1, x, wl, wr, b):
    return pl.pallas_call(
        _pre_body,
        grid=(GRID,),
        in_specs=[_row_spec(HID), _row_spec(HID), _row_spec(16), _row_spec(16),
                  _row_spec(HID), _full_spec(HID, HID), _full_spec(HID, HID),
                  _full_spec(1, HID)],
        out_specs=[_row_spec(HID), _full_spec(8, HID)],
        out_shape=[jax.ShapeDtypeStruct((N, HID), _f32),
                   jax.ShapeDtypeStruct((8, HID), _f32)],
    )(p0, p1, c0, c1, x, wl, wr, b)


def _layer_post(pre, stats, g, bb, cb):
    return pl.pallas_call(
        _post_body,
        grid=(GRID,),
        in_specs=[_row_spec(HID), _full_spec(8, HID), _full_spec(1, HID),
                  _full_spec(1, HID), _full_spec(GROUPS * CODES, HID)],
        out_specs=[_row_spec(HID), _row_spec(8), _full_spec(8, HID)],
        out_shape=[jax.ShapeDtypeStruct((N, HID), _f32),
                   jax.ShapeDtypeStruct((N, 8), jnp.int32),
                   jax.ShapeDtypeStruct((8, HID), _f32)],
    )(pre, stats, g, bb, cb)


def _layer_final(p0, p1, c0, c1, h2, wl, wr, b, cb, w, bl):
    return pl.pallas_call(
        _final_body,
        grid=(GRID,),
        in_specs=[_row_spec(HID), _row_spec(HID), _row_spec(16), _row_spec(16),
                  _row_spec(HID), _full_spec(HID, HID), _full_spec(HID, HID),
                  _full_spec(1, HID), _full_spec(GROUPS * CODES, HID),
                  _full_spec(HID, OUT_C), _full_spec(1, OUT_C)],
        out_specs=[_row_spec(OUT_C), _row_spec(8), _full_spec(8, HID)],
        out_shape=[jax.ShapeDtypeStruct((N, OUT_C), _f32),
                   jax.ShapeDtypeStruct((N, 8), jnp.int32),
                   jax.ShapeDtypeStruct((8, HID), _f32)],
    )(p0, p1, c0, c1, h2, wl, wr, b, cb, w, bl)


def kernel(x, edge_index, params):
    src = edge_index[0]
    dst = edge_index[1]
    src_r = jnp.concatenate(
        [src, jnp.zeros((EPAD - E,), jnp.int32)]).reshape(NW, NCHUNK, CH)
    dst_r = jnp.concatenate(
        [dst, jnp.full((EPAD - E,), DUMMY_ROW, jnp.int32)]).reshape(NW, NCHUNK, CH)

    convs = params['convs']
    bns = params['bns']
    cbs = [cb.reshape(GROUPS * CODES, HID) for cb in params['codebooks']]

    cnts = _sc_degree(dst_r)
    c0 = cnts[0, :N, :16]
    c1 = cnts[1, :N, :16]
    sums = _sc_segment_sum(x, src_r, dst_r)

    h = x
    loss_sums = []
    id_list = []
    for i in range(2):
        pre, stats = _layer_pre(
            sums[0, :N], sums[1, :N], c0, c1, h,
            convs[i]['Wl'], convs[i]['Wr'], convs[i]['b'][None, :])
        h, ids, lacc = _layer_post(
            pre, stats, bns[i]['g'][None, :], bns[i]['b'][None, :], cbs[i])
        id_list.append(ids[:, :GROUPS])
        loss_sums.append(lacc[0, 0])
        sums = _sc_segment_sum(h, src_r, dst_r)

    logits, ids3, lacc3 = _layer_final(
        sums[0, :N], sums[1, :N], c0, c1, h,
        convs[2]['Wl'], convs[2]['Wr'], convs[2]['b'][None, :], cbs[2],
        params['lin']['W'], params['lin']['b'][None, :])
    id_list.append(ids3[:, :GROUPS])
    loss_sums.append(lacc3[0, 0])

    total_loss = (loss_sums[0] + loss_sums[1] + loss_sums[2]) / (N * HID)
    return logits, total_loss, jnp.concatenate(id_list, axis=1)
